# baseline (speedup 1.0000x reference)
import functools

import jax
import jax.numpy as jnp
from jax import lax
from jax.experimental import pallas as pl
from jax.experimental.pallas import tpu as pltpu
from jax.experimental.pallas import tpu_sc as plsc

NC = 2
NS = 16
NW = NC * NS

IDXW = 128
KPER = 8
CHUNK = IDXW * KPER


def _sc_gather(table, idx2):
    n_idx_rows, _ = idx2.shape
    b_tot = n_idx_rows * IDXW
    h = table.shape[1]
    b_per_w = b_tot // NW
    n_chunks = b_per_w // CHUNK

    mesh = plsc.VectorSubcoreMesh(core_axis_name="c", subcore_axis_name="s")

    @functools.partial(
        pl.kernel,
        mesh=mesh,
        compiler_params=pltpu.CompilerParams(use_tc_tiling_on_sc=False),
        out_type=jax.ShapeDtypeStruct((b_tot, h), jnp.float32),
        scratch_types=[
            pltpu.VMEM((KPER, IDXW), jnp.int32),
            pltpu.VMEM((CHUNK, h), jnp.float32),
            pltpu.SemaphoreType.DMA,
        ],
    )
    def k(table_hbm, idx_hbm, out_hbm, idx_v, rows_v, sem):
        wid = lax.axis_index("s") * NC + lax.axis_index("c")
        base = wid * b_per_w

        @pl.loop(0, n_chunks)
        def _(c):
            off = pl.multiple_of(base + c * CHUNK, CHUNK)
            idx_row = pl.multiple_of(off // IDXW, KPER)
            pltpu.sync_copy(idx_hbm.at[pl.ds(idx_row, KPER)], idx_v)
            copies = []
            for j in range(KPER):
                copies.append(pltpu.async_copy(
                    table_hbm.at[idx_v.at[j]],
                    rows_v.at[pl.ds(j * IDXW, IDXW)],
                    sem,
                ))
            for cp in copies:
                cp.wait()
            pltpu.sync_copy(rows_v, out_hbm.at[pl.ds(off, CHUNK)])

    return k(table, idx2)


def _ln_body(g_ref, p_ref, gam_ref, bet_ref, o_ref):
    x = g_ref[...] + p_ref[...]
    m = jnp.mean(x, axis=-1, keepdims=True)
    d = x - m
    v = jnp.mean(d * d, axis=-1, keepdims=True)
    o_ref[...] = d * lax.rsqrt(v + 1e-12) * gam_ref[...] + bet_ref[...]


def _tc_ln(gathered3, pos3, gamma3, beta3):
    b, s, h = gathered3.shape
    bb = 64
    grid = (b // bb,)
    return pl.pallas_call(
        _ln_body,
        grid=grid,
        in_specs=[
            pl.BlockSpec((bb, s, h), lambda i: (i, 0, 0)),
            pl.BlockSpec((1, s, h), lambda i: (0, 0, 0)),
            pl.BlockSpec((1, 1, h), lambda i: (0, 0, 0)),
            pl.BlockSpec((1, 1, h), lambda i: (0, 0, 0)),
        ],
        out_specs=pl.BlockSpec((bb, s, h), lambda i: (i, 0, 0)),
        out_shape=jax.ShapeDtypeStruct((b, s, h), jnp.float32),
    )(gathered3, pos3, gamma3, beta3)


def kernel(input_ids, entry_emb, pos_emb, ln_gamma, ln_beta):
    b, s = input_ids.shape
    h = entry_emb.shape[1]
    idx2 = input_ids.reshape(-1, IDXW)
    gathered = _sc_gather(entry_emb, idx2)
    return _tc_ln(
        gathered.reshape(b, s, h),
        pos_emb[:s][None],
        ln_gamma.reshape(1, 1, h),
        ln_beta.reshape(1, 1, h),
    )


# speedup vs baseline: 1.0034x; 1.0034x over previous
import functools

import jax
import jax.numpy as jnp
from jax import lax
from jax.experimental import pallas as pl
from jax.experimental.pallas import tpu as pltpu
from jax.experimental.pallas import tpu_sc as plsc

NC = 2
NS = 16
NW = NC * NS

IDXW = 128
KPER = 8
CHUNK = IDXW * KPER


def _sc_gather(table, idx2):
    n_idx_rows, _ = idx2.shape
    b_tot = n_idx_rows * IDXW
    h = table.shape[1]
    b_per_w = b_tot // NW
    n_chunks = b_per_w // CHUNK

    mesh = plsc.VectorSubcoreMesh(core_axis_name="c", subcore_axis_name="s")

    @functools.partial(
        pl.kernel,
        mesh=mesh,
        compiler_params=pltpu.CompilerParams(use_tc_tiling_on_sc=False),
        out_type=jax.ShapeDtypeStruct((b_tot, h), jnp.float32),
        scratch_types=[
            pltpu.VMEM((KPER, IDXW), jnp.int32),
            pltpu.VMEM((CHUNK, h), jnp.float32),
            pltpu.SemaphoreType.DMA,
        ],
    )
    def k(table_hbm, idx_hbm, out_hbm, idx_v, rows_v, sem):
        wid = lax.axis_index("s") * NC + lax.axis_index("c")
        base = wid * b_per_w

        @pl.loop(0, n_chunks)
        def _(c):
            off = pl.multiple_of(base + c * CHUNK, CHUNK)
            idx_row = pl.multiple_of(off // IDXW, KPER)
            pltpu.sync_copy(idx_hbm.at[pl.ds(idx_row, KPER)], idx_v)
            copies = []
            for j in range(KPER):
                copies.append(pltpu.async_copy(
                    table_hbm.at[idx_v.at[j]],
                    rows_v.at[pl.ds(j * IDXW, IDXW)],
                    sem,
                ))
            for cp in copies:
                cp.wait()
            pltpu.sync_copy(rows_v, out_hbm.at[pl.ds(off, CHUNK)])

    return k(table, idx2)


def _ln_body(g_ref, p_ref, gam_ref, bet_ref, o_ref):
    x = g_ref[...].reshape(o_ref.shape) + p_ref[...]
    m = jnp.mean(x, axis=-1, keepdims=True)
    d = x - m
    v = jnp.mean(d * d, axis=-1, keepdims=True)
    o_ref[...] = d * lax.rsqrt(v + 1e-12) * gam_ref[...] + bet_ref[...]


def _tc_ln(gathered2, pos3, gamma3, beta3, b, s, h):
    bb = 64
    grid = (b // bb,)
    return pl.pallas_call(
        _ln_body,
        grid=grid,
        in_specs=[
            pl.BlockSpec((bb * s, h), lambda i: (i, 0)),
            pl.BlockSpec((1, s, h), lambda i: (0, 0, 0)),
            pl.BlockSpec((1, 1, h), lambda i: (0, 0, 0)),
            pl.BlockSpec((1, 1, h), lambda i: (0, 0, 0)),
        ],
        out_specs=pl.BlockSpec((bb, s, h), lambda i: (i, 0, 0)),
        out_shape=jax.ShapeDtypeStruct((b, s, h), jnp.float32),
    )(gathered2, pos3, gamma3, beta3)


def kernel(input_ids, entry_emb, pos_emb, ln_gamma, ln_beta):
    b, s = input_ids.shape
    h = entry_emb.shape[1]
    idx2 = input_ids.reshape(-1, IDXW)
    gathered = _sc_gather(entry_emb, idx2)
    return _tc_ln(
        gathered,
        pos_emb[:s][None],
        ln_gamma.reshape(1, 1, h),
        ln_beta.reshape(1, 1, h),
        b, s, h,
    )
